# baseline (speedup 1.0000x reference)
import functools

import jax
import jax.numpy as jnp
from jax import lax
from jax.experimental import pallas as pl
from jax.experimental.pallas import tpu as pltpu
from jax.experimental.pallas import tpu_sc as plsc

VOCAB = 1000000
DIM = 64
B_TOK = 16384
SEQ = 50

NC = 2
NS = 16
NW = NC * NS

R_PER_W = B_TOK // NW
CH = 4
NCHUNK = R_PER_W // CH
NBUF = 4
NGRP = NCHUNK // NBUF

_mesh = plsc.VectorSubcoreMesh(core_axis_name="c", subcore_axis_name="s")


@functools.partial(
    pl.kernel,
    mesh=_mesh,
    out_type=jax.ShapeDtypeStruct((B_TOK, SEQ, DIM), jnp.float32),
    scratch_types=[
        pltpu.VMEM((R_PER_W, SEQ), jnp.int32),
        pltpu.VMEM((NBUF, CH, SEQ, DIM), jnp.float32),
        pltpu.SemaphoreType.DMA((NBUF,)),
        pltpu.SemaphoreType.DMA((NBUF,)),
    ],
    compiler_params=pltpu.CompilerParams(use_tc_tiling_on_sc=False),
)
def _emb_lookup(ind_hbm, weight_hbm, out_hbm, idx_v, rows_v, gsem, wsem):
    wid = lax.axis_index("s") * NC + lax.axis_index("c")
    row0 = wid * R_PER_W
    pltpu.sync_copy(ind_hbm.at[pl.ds(row0, R_PER_W)], idx_v)

    def start_gathers(chunk, b):
        for j in range(CH):
            pltpu.async_copy(
                weight_hbm.at[idx_v.at[chunk * CH + j]],
                rows_v.at[b].at[j], gsem.at[b])

    def wait_gathers(b):
        for j in range(CH):
            pltpu.make_async_copy(
                weight_hbm.at[idx_v.at[0]],
                rows_v.at[b].at[j], gsem.at[b]).wait()

    def wait_write(b):
        pltpu.make_async_copy(
            rows_v.at[b], out_hbm.at[pl.ds(row0, CH)], wsem.at[b]).wait()

    for b in range(NBUF):
        start_gathers(b, b)

    def group(i, carry):
        for b in range(NBUF):
            wait_gathers(b)
            pltpu.async_copy(
                rows_v.at[b],
                out_hbm.at[pl.ds(row0 + (i * NBUF + b) * CH, CH)],
                wsem.at[b])

        @pl.when(i < NGRP - 1)
        def _():
            for b in range(NBUF):
                wait_write(b)
                start_gathers((i + 1) * NBUF + b, b)

        return carry

    lax.fori_loop(0, NGRP, group, 0)
    for b in range(NBUF):
        wait_write(b)


def kernel(ind, weight):
    return _emb_lookup(ind, weight)


# speedup vs baseline: 1.0549x; 1.0549x over previous
import functools

import jax
import jax.numpy as jnp
from jax import lax
from jax.experimental import pallas as pl
from jax.experimental.pallas import tpu as pltpu
from jax.experimental.pallas import tpu_sc as plsc

VOCAB = 1000000
DIM = 64
B_TOK = 16384
SEQ = 50

NC = 2
NS = 16
NW = NC * NS

R_PER_W = B_TOK // NW
CH = 4
NCHUNK = R_PER_W // CH
NBUF = 4
NGRP = NCHUNK // NBUF

_mesh = plsc.VectorSubcoreMesh(core_axis_name="c", subcore_axis_name="s")


@functools.partial(
    pl.kernel,
    mesh=_mesh,
    out_type=jax.ShapeDtypeStruct((B_TOK, SEQ, DIM), jnp.float32),
    scratch_types=[
        pltpu.VMEM((R_PER_W, SEQ), jnp.int32),
        pltpu.VMEM((NBUF, CH, SEQ, DIM), jnp.float32),
        pltpu.SemaphoreType.DMA((NBUF,)),
        pltpu.SemaphoreType.DMA((NBUF,)),
    ],
    compiler_params=pltpu.CompilerParams(use_tc_tiling_on_sc=False),
)
def _emb_lookup(ind_hbm, weight_hbm, out_hbm, idx_v, rows_v, gsem, wsem):
    wid = lax.axis_index("s") * NC + lax.axis_index("c")
    row0 = wid * R_PER_W
    pltpu.sync_copy(ind_hbm.at[pl.ds(row0, R_PER_W)], idx_v)

    def start_gathers(chunk, b):
        for j in range(CH):
            pltpu.async_copy(
                weight_hbm.at[idx_v.at[chunk * CH + j]],
                rows_v.at[b].at[j], gsem.at[b])

    def wait_gathers(b):
        for j in range(CH):
            pltpu.make_async_copy(
                weight_hbm.at[idx_v.at[0]],
                rows_v.at[b].at[j], gsem.at[b]).wait()

    def wait_write(b):
        pltpu.make_async_copy(
            rows_v.at[b], out_hbm.at[pl.ds(row0, CH)], wsem.at[b]).wait()

    for b in range(NBUF):
        start_gathers(b, b)

    def group(i, carry):
        for b in range(NBUF):
            wait_gathers(b)
            pltpu.async_copy(
                rows_v.at[b],
                out_hbm.at[pl.ds(row0 + (i * NBUF + b) * CH, CH)],
                wsem.at[b])

        @pl.when(i < NGRP - 1)
        def _():
            for b in range(NBUF):
                wait_write(b)
                start_gathers((i + 1) * NBUF + b, b)

        return carry

    lax.fori_loop(0, NGRP, group, 0)
    for b in range(NBUF):
        wait_write(b)


def kernel(ind, weight):
    wpad = jnp.pad(weight, ((0, 0), (0, 64)))
    w2 = wpad.reshape(2 * VOCAB, DIM)
    return _emb_lookup(ind * 2, w2)


# speedup vs baseline: 1.2368x; 1.1724x over previous
import functools

import jax
import jax.numpy as jnp
from jax import lax
from jax.experimental import pallas as pl
from jax.experimental.pallas import tpu as pltpu
from jax.experimental.pallas import tpu_sc as plsc

VOCAB = 1000000
DIM = 64
DIMP = 128
B_TOK = 16384
SEQ = 50
SEQP = 56

NC = 2
NS = 16
NW = NC * NS

R_PER_W = B_TOK // NW
CH = 2
NCHUNK = R_PER_W // CH
NBUF = 4
NGRP = NCHUNK // NBUF

_mesh = plsc.VectorSubcoreMesh(core_axis_name="c", subcore_axis_name="s")


@functools.partial(
    pl.kernel,
    mesh=_mesh,
    out_type=jax.ShapeDtypeStruct((B_TOK, SEQP, DIMP), jnp.float32),
    scratch_types=[
        pltpu.VMEM((R_PER_W, SEQ), jnp.int32),
        pltpu.VMEM((NBUF, CH, SEQ, DIMP), jnp.float32),
        pltpu.SemaphoreType.DMA((NBUF,)),
        pltpu.SemaphoreType.DMA((NBUF,)),
    ],
    compiler_params=pltpu.CompilerParams(use_tc_tiling_on_sc=False),
)
def _emb_lookup(ind_hbm, wpad_hbm, out_hbm, idx_v, rows_v, gsem, wsem):
    wid = lax.axis_index("s") * NC + lax.axis_index("c")
    row0 = wid * R_PER_W
    pltpu.sync_copy(ind_hbm.at[pl.ds(row0, R_PER_W)], idx_v)

    def start_gathers(chunk, b):
        for j in range(CH):
            pltpu.async_copy(
                wpad_hbm.at[idx_v.at[chunk * CH + j]],
                rows_v.at[b].at[j], gsem.at[b])

    def wait_gathers(b):
        for j in range(CH):
            pltpu.make_async_copy(
                wpad_hbm.at[idx_v.at[0]],
                rows_v.at[b].at[j], gsem.at[b]).wait()

    def start_writes(chunk, b):
        for j in range(CH):
            pltpu.async_copy(
                rows_v.at[b].at[j],
                out_hbm.at[row0 + chunk * CH + j].at[pl.ds(0, SEQ)],
                wsem.at[b])

    def wait_writes(b):
        for j in range(CH):
            pltpu.make_async_copy(
                rows_v.at[b].at[j],
                out_hbm.at[row0].at[pl.ds(0, SEQ)], wsem.at[b]).wait()

    for b in range(NBUF):
        start_gathers(b, b)

    def group(i, carry):
        for b in range(NBUF):
            wait_gathers(b)
            start_writes(i * NBUF + b, b)

        @pl.when(i < NGRP - 1)
        def _():
            for b in range(NBUF):
                wait_writes(b)
                start_gathers((i + 1) * NBUF + b, b)

        return carry

    lax.fori_loop(0, NGRP, group, 0)
    for b in range(NBUF):
        wait_writes(b)


def kernel(ind, weight):
    wpad = jnp.pad(weight, ((0, 0), (0, DIMP - DIM)))
    outp = _emb_lookup(ind, wpad)
    return outp[:, :SEQ, :DIM]


# speedup vs baseline: 1.4472x; 1.1702x over previous
import functools

import jax
import jax.numpy as jnp
from jax import lax
from jax.experimental import pallas as pl
from jax.experimental.pallas import tpu as pltpu
from jax.experimental.pallas import tpu_sc as plsc

VOCAB = 1000000
DIM = 64
DIMP = 128
B_TOK = 16384
SEQ = 50
SEQP = 56

NC = 2
NS = 16
NW = NC * NS

R_PER_W = B_TOK // NW
CH = 4
NCHUNK = R_PER_W // CH
NBUF = 4
NGRP = NCHUNK // NBUF

_mesh = plsc.VectorSubcoreMesh(core_axis_name="c", subcore_axis_name="s")


@functools.partial(
    pl.kernel,
    mesh=_mesh,
    out_type=jax.ShapeDtypeStruct((B_TOK, SEQP, DIMP), jnp.float32),
    scratch_types=[
        pltpu.VMEM((R_PER_W, SEQ), jnp.int32),
        pltpu.VMEM((NBUF, CH, SEQ, DIM), jnp.float32),
        pltpu.SemaphoreType.DMA((NBUF,)),
        pltpu.SemaphoreType.DMA((NBUF,)),
    ],
    compiler_params=pltpu.CompilerParams(use_tc_tiling_on_sc=False),
)
def _emb_lookup(ind_hbm, wpad_hbm, out_hbm, idx_v, rows_v, gsem, wsem):
    wid = lax.axis_index("s") * NC + lax.axis_index("c")
    row0 = wid * R_PER_W
    pltpu.sync_copy(ind_hbm.at[pl.ds(row0, R_PER_W)], idx_v)

    def start_gathers(chunk, b):
        for j in range(CH):
            pltpu.async_copy(
                wpad_hbm.at[idx_v.at[chunk * CH + j]],
                rows_v.at[b].at[j], gsem.at[b])

    def wait_gathers(b):
        for j in range(CH):
            pltpu.make_async_copy(
                wpad_hbm.at[idx_v.at[0]],
                rows_v.at[b].at[j], gsem.at[b]).wait()

    def start_writes(chunk, b):
        for j in range(CH):
            pltpu.async_copy(
                rows_v.at[b].at[j],
                out_hbm.at[row0 + chunk * CH + j].at[pl.ds(0, SEQ), pl.ds(0, DIM)],
                wsem.at[b])

    def wait_writes(b):
        for j in range(CH):
            pltpu.make_async_copy(
                rows_v.at[b].at[j],
                out_hbm.at[row0].at[pl.ds(0, SEQ), pl.ds(0, DIM)],
                wsem.at[b]).wait()

    for b in range(NBUF):
        start_gathers(b, b)

    def group(i, carry):
        for b in range(NBUF):
            wait_gathers(b)
            start_writes(i * NBUF + b, b)

        @pl.when(i < NGRP - 1)
        def _():
            for b in range(NBUF):
                wait_writes(b)
                start_gathers((i + 1) * NBUF + b, b)

        return carry

    lax.fori_loop(0, NGRP, group, 0)
    for b in range(NBUF):
        wait_writes(b)


def kernel(ind, weight):
    wpad = jnp.pad(weight, ((0, 0), (0, DIMP - DIM)))
    w2 = wpad.reshape(2 * VOCAB, DIM)
    outp = _emb_lookup(ind * 2, w2)
    return outp[:, :SEQ, :DIM]


# speedup vs baseline: 1.8185x; 1.2565x over previous
import functools

import jax
import jax.numpy as jnp
from jax import lax
from jax.experimental import pallas as pl
from jax.experimental.pallas import tpu as pltpu
from jax.experimental.pallas import tpu_sc as plsc

VOCAB = 1000000
DIM = 64
DIMP = 128
B_TOK = 16384
SEQ = 50
SEQP = 56

NC = 2
NS = 16
NW = NC * NS

R_PER_W = B_TOK // NW
CH = 4
NCHUNK = R_PER_W // CH
NBUF = 4
NGRP = NCHUNK // NBUF

TBLK = 4096

_mesh = plsc.VectorSubcoreMesh(core_axis_name="c", subcore_axis_name="s")


def _tpad_body(wt_ref, out_ref):
    out_ref[:, :DIM] = wt_ref[...].T


def _transpose_pad(wt):
    grid = (VOCAB + TBLK - 1) // TBLK
    return pl.pallas_call(
        _tpad_body,
        grid=(grid,),
        in_specs=[pl.BlockSpec((DIM, TBLK), lambda j: (0, j))],
        out_specs=pl.BlockSpec((TBLK, DIMP), lambda j: (j, 0)),
        out_shape=jax.ShapeDtypeStruct((VOCAB, DIMP), jnp.float32),
    )(wt)


@functools.partial(
    pl.kernel,
    mesh=_mesh,
    out_type=jax.ShapeDtypeStruct((B_TOK, SEQP, DIMP), jnp.float32),
    scratch_types=[
        pltpu.VMEM((R_PER_W, SEQ), jnp.int32),
        pltpu.VMEM((NBUF, CH, SEQ, DIM), jnp.float32),
        pltpu.SemaphoreType.DMA((NBUF,)),
        pltpu.SemaphoreType.DMA((NBUF,)),
    ],
    compiler_params=pltpu.CompilerParams(use_tc_tiling_on_sc=False),
)
def _emb_lookup(ind_hbm, wpad_hbm, out_hbm, idx_v, rows_v, gsem, wsem):
    wid = lax.axis_index("s") * NC + lax.axis_index("c")
    row0 = wid * R_PER_W
    pltpu.sync_copy(ind_hbm.at[pl.ds(row0, R_PER_W)], idx_v)

    def start_gathers(chunk, b):
        for j in range(CH):
            pltpu.async_copy(
                wpad_hbm.at[idx_v.at[chunk * CH + j]],
                rows_v.at[b].at[j], gsem.at[b])

    def wait_gathers(b):
        for j in range(CH):
            pltpu.make_async_copy(
                wpad_hbm.at[idx_v.at[0]],
                rows_v.at[b].at[j], gsem.at[b]).wait()

    def start_writes(chunk, b):
        for j in range(CH):
            pltpu.async_copy(
                rows_v.at[b].at[j],
                out_hbm.at[row0 + chunk * CH + j].at[pl.ds(0, SEQ), pl.ds(0, DIM)],
                wsem.at[b])

    def wait_writes(b):
        for j in range(CH):
            pltpu.make_async_copy(
                rows_v.at[b].at[j],
                out_hbm.at[row0].at[pl.ds(0, SEQ), pl.ds(0, DIM)],
                wsem.at[b]).wait()

    for b in range(NBUF):
        start_gathers(b, b)

    def group(i, carry):
        for b in range(NBUF):
            wait_gathers(b)
            start_writes(i * NBUF + b, b)

        @pl.when(i < NGRP - 1)
        def _():
            for b in range(NBUF):
                wait_writes(b)
                start_gathers((i + 1) * NBUF + b, b)

        return carry

    lax.fori_loop(0, NGRP, group, 0)
    for b in range(NBUF):
        wait_writes(b)


def kernel(ind, weight):
    wpad = _transpose_pad(weight.T)
    w2 = wpad.reshape(2 * VOCAB, DIM)
    outp = _emb_lookup(ind * 2, w2)
    return outp[:, :SEQ, :DIM]


# speedup vs baseline: 2.0323x; 1.1176x over previous
import functools

import jax
import jax.numpy as jnp
from jax import lax
from jax.experimental import pallas as pl
from jax.experimental.pallas import tpu as pltpu
from jax.experimental.pallas import tpu_sc as plsc

VOCAB = 1000000
DIM = 64
DIMP = 128
B_TOK = 16384
SEQ = 50
SEQP = 56

NC = 2
NS = 16
NW = NC * NS

R_PER_W = B_TOK // NW
CH = 4
NCHUNK = R_PER_W // CH
NBUF = 4
NGRP = NCHUNK // NBUF

TBLK = 8192

_mesh = plsc.VectorSubcoreMesh(core_axis_name="c", subcore_axis_name="s")


def _tpad_body(wt_ref, out_ref):
    out_ref[:, :DIM] = wt_ref[...].T


def _transpose_pad(wt):
    grid = (VOCAB + TBLK - 1) // TBLK
    return pl.pallas_call(
        _tpad_body,
        grid=(grid,),
        in_specs=[pl.BlockSpec((DIM, TBLK), lambda j: (0, j))],
        out_specs=pl.BlockSpec((TBLK, DIMP), lambda j: (j, 0)),
        out_shape=jax.ShapeDtypeStruct((VOCAB, DIMP), jnp.float32),
    )(wt)


@functools.partial(
    pl.kernel,
    mesh=_mesh,
    out_type=jax.ShapeDtypeStruct((B_TOK, SEQP, DIMP), jnp.float32),
    scratch_types=[
        pltpu.VMEM((R_PER_W, SEQ), jnp.int32),
        pltpu.VMEM((NBUF, CH, SEQ, DIM), jnp.float32),
        pltpu.SemaphoreType.DMA((NBUF,)),
        pltpu.SemaphoreType.DMA((NBUF,)),
    ],
    compiler_params=pltpu.CompilerParams(use_tc_tiling_on_sc=False),
)
def _emb_lookup(ind_hbm, wpad_hbm, out_hbm, idx_v, rows_v, gsem, wsem):
    wid = lax.axis_index("s") * NC + lax.axis_index("c")
    row0 = wid * R_PER_W
    pltpu.sync_copy(ind_hbm.at[pl.ds(row0, R_PER_W)], idx_v)

    def start_gathers(chunk, b):
        for j in range(CH):
            pltpu.async_copy(
                wpad_hbm.at[idx_v.at[chunk * CH + j]],
                rows_v.at[b].at[j], gsem.at[b])

    def wait_gathers(b):
        for j in range(CH):
            pltpu.make_async_copy(
                wpad_hbm.at[idx_v.at[0]],
                rows_v.at[b].at[j], gsem.at[b]).wait()

    def start_writes(chunk, b):
        for j in range(CH):
            pltpu.async_copy(
                rows_v.at[b].at[j],
                out_hbm.at[row0 + chunk * CH + j].at[pl.ds(0, SEQ), pl.ds(0, DIM)],
                wsem.at[b])

    def wait_writes(b):
        for j in range(CH):
            pltpu.make_async_copy(
                rows_v.at[b].at[j],
                out_hbm.at[row0].at[pl.ds(0, SEQ), pl.ds(0, DIM)],
                wsem.at[b]).wait()

    for b in range(NBUF):
        start_gathers(b, b)

    def group(i, carry):
        for b in range(NBUF):
            wait_gathers(b)
            start_writes(i * NBUF + b, b)

        @pl.when(i < NGRP - 1)
        def _():
            for b in range(NBUF):
                wait_writes(b)
                start_gathers((i + 1) * NBUF + b, b)

        return carry

    lax.fori_loop(0, NGRP, group, 0)
    for b in range(NBUF):
        wait_writes(b)


def kernel(ind, weight):
    wpad = _transpose_pad(weight.T)
    w2 = wpad.reshape(2 * VOCAB, DIM)
    outp = _emb_lookup(ind * 2, w2)
    return outp[:, :SEQ, :DIM]


# speedup vs baseline: 2.0946x; 1.0307x over previous
import functools

import jax
import jax.numpy as jnp
from jax import lax
from jax.experimental import pallas as pl
from jax.experimental.pallas import tpu as pltpu
from jax.experimental.pallas import tpu_sc as plsc

VOCAB = 1000000
DIM = 64
DIMP = 128
B_TOK = 16384
SEQ = 50
SEQP = 56

NC = 2
NS = 16
NW = NC * NS

R_PER_W = B_TOK // NW
CH = 4
NCHUNK = R_PER_W // CH
NBUF = 4
NGRP = NCHUNK // NBUF

TBLK = 16384

_mesh = plsc.VectorSubcoreMesh(core_axis_name="c", subcore_axis_name="s")


def _tpad_body(wt_ref, out_ref):
    out_ref[:, :DIM] = wt_ref[...].T


def _transpose_pad(wt):
    grid = (VOCAB + TBLK - 1) // TBLK
    return pl.pallas_call(
        _tpad_body,
        grid=(grid,),
        in_specs=[pl.BlockSpec((DIM, TBLK), lambda j: (0, j))],
        out_specs=pl.BlockSpec((TBLK, DIMP), lambda j: (j, 0)),
        out_shape=jax.ShapeDtypeStruct((VOCAB, DIMP), jnp.float32),
    )(wt)


@functools.partial(
    pl.kernel,
    mesh=_mesh,
    out_type=jax.ShapeDtypeStruct((B_TOK, SEQP, DIMP), jnp.float32),
    scratch_types=[
        pltpu.VMEM((R_PER_W, SEQ), jnp.int32),
        pltpu.VMEM((NBUF, CH, SEQ, DIM), jnp.float32),
        pltpu.SemaphoreType.DMA((NBUF,)),
        pltpu.SemaphoreType.DMA((NBUF,)),
    ],
    compiler_params=pltpu.CompilerParams(use_tc_tiling_on_sc=False),
)
def _emb_lookup(ind_hbm, wpad_hbm, out_hbm, idx_v, rows_v, gsem, wsem):
    wid = lax.axis_index("s") * NC + lax.axis_index("c")
    row0 = wid * R_PER_W
    pltpu.sync_copy(ind_hbm.at[pl.ds(row0, R_PER_W)], idx_v)

    def start_gathers(chunk, b):
        for j in range(CH):
            pltpu.async_copy(
                wpad_hbm.at[idx_v.at[chunk * CH + j]],
                rows_v.at[b].at[j], gsem.at[b])

    def wait_gathers(b):
        for j in range(CH):
            pltpu.make_async_copy(
                wpad_hbm.at[idx_v.at[0]],
                rows_v.at[b].at[j], gsem.at[b]).wait()

    def start_writes(chunk, b):
        for j in range(CH):
            pltpu.async_copy(
                rows_v.at[b].at[j],
                out_hbm.at[row0 + chunk * CH + j].at[pl.ds(0, SEQ), pl.ds(0, DIM)],
                wsem.at[b])

    def wait_writes(b):
        for j in range(CH):
            pltpu.make_async_copy(
                rows_v.at[b].at[j],
                out_hbm.at[row0].at[pl.ds(0, SEQ), pl.ds(0, DIM)],
                wsem.at[b]).wait()

    for b in range(NBUF):
        start_gathers(b, b)

    def group(i, carry):
        for b in range(NBUF):
            wait_gathers(b)
            start_writes(i * NBUF + b, b)

        @pl.when(i < NGRP - 1)
        def _():
            for b in range(NBUF):
                wait_writes(b)
                start_gathers((i + 1) * NBUF + b, b)

        return carry

    lax.fori_loop(0, NGRP, group, 0)
    for b in range(NBUF):
        wait_writes(b)


def kernel(ind, weight):
    wpad = _transpose_pad(weight.T)
    w2 = wpad.reshape(2 * VOCAB, DIM)
    outp = _emb_lookup(ind * 2, w2)
    return outp[:, :SEQ, :DIM]
